# baseline (speedup 1.0000x reference)
import functools

import jax
import jax.numpy as jnp
from jax import lax
from jax.experimental import pallas as pl
from jax.experimental.pallas import tpu as pltpu
from jax.experimental.pallas import tpu_sc as plsc

_D = 64
_SCALE = float(_D) ** 0.5
_NC = 2
_NS = 16
_NW = _NC * _NS
_LANES = 16
_NB = 4
_BLK = 128


@functools.partial(jax.jit, static_argnums=(2,))
def _lookup(x_flat, table2, per_w):
  n_c = per_w // _BLK
  mesh = plsc.VectorSubcoreMesh(
      core_axis_name="c", subcore_axis_name="s", num_cores=_NC,
      num_subcores=_NS)

  @functools.partial(
      pl.kernel,
      mesh=mesh,
      out_type=jax.ShapeDtypeStruct((x_flat.shape[0] // 2, 2 * _D),
                                    jnp.float32),
      scratch_types=[
          pltpu.VMEM((per_w,), jnp.int32),
          [pltpu.VMEM((_BLK,), jnp.int32) for _ in range(_NB)],
          [pltpu.VMEM((_BLK, 2 * _D), jnp.float32) for _ in range(_NB)],
          [pltpu.VMEM((_BLK // 2, 2 * _D), jnp.float32) for _ in range(_NB)],
          [pltpu.SemaphoreType.DMA for _ in range(_NB)],
          [pltpu.SemaphoreType.DMA for _ in range(_NB)],
      ],
      compiler_params=pltpu.CompilerParams(
          use_tc_tiling_on_sc=True, needs_layout_passes=False),
  )
  def body(x_hbm, tab_hbm, out_hbm, idx_v, jrow, rows, outb, sem_g, sem_s):
    wid = lax.axis_index("s") * _NC + lax.axis_index("c")
    base = pl.multiple_of(wid * per_w, per_w)
    pltpu.sync_copy(x_hbm.at[pl.ds(base, per_w)], idx_v)

    def start_gather(c, b):
      for k in range(_BLK // _LANES):
        sl = pl.ds(c * _BLK + k * _LANES, _LANES)
        jrow[b][pl.ds(k * _LANES, _LANES)] = lax.shift_right_logical(
            idx_v[sl], 1)
      pltpu.make_async_copy(
          tab_hbm.at[jrow[b]], rows[b], sem_g[b]).start()

    def wait_gather(b):
      pltpu.make_async_copy(
          tab_hbm.at[jrow[b]], rows[b], sem_g[b]).wait()

    def scatter(c, b):
      return pltpu.make_async_copy(
          outb[b],
          out_hbm.at[
              pl.ds(pl.multiple_of((base + c * _BLK) // 2, _BLK // 2),
                    _BLK // 2), :],
          sem_s[b])

    for b in range(_NB - 1):
      start_gather(b, b)

    def group_body(go, carry):
      for b in range(_NB):
        c = go * _NB + b
        wait_gather(b)

        @plsc.parallel_loop(0, _BLK // _LANES)
        def _(rb):
          hv = (idx_v[pl.ds(c * _BLK + rb * _LANES, _LANES)] & 1) * _D
          for ri in range(_LANES):
            src0 = hv[ri]
            for k in range(_D // _LANES):
              v = rows[b][rb * _LANES + ri,
                          pl.ds(src0 + k * _LANES, _LANES)]
              outb[b][rb * (_LANES // 2) + ri // 2,
                      pl.ds((ri & 1) * _D + k * _LANES, _LANES)] = v * _SCALE

        scatter(c, b).start()

        nb = (b + _NB - 1) % _NB
        nc = c + _NB - 1

        @pl.when(nc < n_c)
        def _():
          @pl.when(c >= 1)
          def _():
            scatter(c - 1, nb).wait()
          start_gather(nc, nb)

      return carry

    lax.fori_loop(0, n_c // _NB, group_body, 0)
    for b in range(_NB):
      scatter(n_c - _NB + b, b).wait()

  return body(x_flat, table2)


def kernel(x, table):
  b, l = x.shape
  n = b * l
  x_flat = jnp.reshape(x, (n,)).astype(jnp.int32)
  table2 = jnp.reshape(table, (-1, 2 * _D))
  packed = _lookup(x_flat, table2, n // _NW)
  return jnp.reshape(packed, (b, l, _D))


# speedup vs baseline: 1.1158x; 1.1158x over previous
import functools

import jax
import jax.numpy as jnp
from jax import lax
from jax.experimental import pallas as pl
from jax.experimental.pallas import tpu as pltpu
from jax.experimental.pallas import tpu_sc as plsc

_D = 64
_SCALE = float(_D) ** 0.5
_NC = 2
_NS = 16
_NW = _NC * _NS
_LANES = 16
_NB = 4


@functools.partial(jax.jit, static_argnums=(2, 3))
def _lookup(x_flat, table_flat, per_w, chunk):
  table = jnp.reshape(table_flat, (-1, _D))
  n_chunks = per_w // chunk
  assert n_chunks % _NB == 0
  mesh = plsc.VectorSubcoreMesh(
      core_axis_name="c", subcore_axis_name="s", num_cores=_NC,
      num_subcores=_NS)

  @functools.partial(
      pl.kernel,
      mesh=mesh,
      out_type=jax.ShapeDtypeStruct((x_flat.shape[0], _D), jnp.float32),
      scratch_types=[
          pltpu.VMEM((per_w,), jnp.int32),
          [pltpu.VMEM((chunk, _D), jnp.float32) for _ in range(_NB)],
          [pltpu.SemaphoreType.DMA for _ in range(_NB)],
          [pltpu.SemaphoreType.DMA for _ in range(_NB)],
      ],
      compiler_params=pltpu.CompilerParams(use_tc_tiling_on_sc=False),
  )
  def body(x_hbm, table_hbm, out_hbm, idx_v, rows, sem_g, sem_s):
    wid = lax.axis_index("s") * _NC + lax.axis_index("c")
    base = wid * per_w
    pltpu.sync_copy(x_hbm.at[pl.ds(base, per_w)], idx_v)

    def gather(c, b):
      return pltpu.make_async_copy(
          table_hbm.at[idx_v.at[pl.ds(c * chunk, chunk)]], rows[b], sem_g[b])

    def scatter(c, b):
      return pltpu.make_async_copy(
          rows[b], out_hbm.at[pl.ds(base + c * chunk, chunk)], sem_s[b])

    for b in range(_NB - 1):
      gather(b, b).start()

    def group_body(go, carry):
      for b in range(_NB):
        c = go * _NB + b
        gather(c, b).wait()

        def scale_body(r, acc):
          for j in range(_D // _LANES):
            sl = pl.ds(j * _LANES, _LANES)
            rows[b][r, sl] = rows[b][r, sl] * _SCALE
          return acc

        lax.fori_loop(0, chunk, scale_body, 0, unroll=4)
        scatter(c, b).start()

        nb = (b + _NB - 1) % _NB
        nc = c + _NB - 1

        @pl.when(nc < n_chunks)
        def _():
          @pl.when(c >= 1)
          def _():
            scatter(c - 1, nb).wait()
          gather(nc, nb).start()

      return carry

    lax.fori_loop(0, n_chunks // _NB, group_body, 0)
    for b in range(_NB):
      scatter(n_chunks - _NB + b, b).wait()

  return body(x_flat, table)


def kernel(x, table):
  b, l = x.shape
  n = b * l
  per_w = n // _NW
  x_flat = jnp.reshape(x, (n,)).astype(jnp.int32)
  table_flat = jnp.reshape(table, (-1,))
  out = _lookup(x_flat, table_flat, per_w, 128)
  return jnp.reshape(out, (b, l, _D))


# speedup vs baseline: 1.1167x; 1.0008x over previous
import functools

import jax
import jax.numpy as jnp
from jax import lax
from jax.experimental import pallas as pl
from jax.experimental.pallas import tpu as pltpu
from jax.experimental.pallas import tpu_sc as plsc

_D = 64
_SCALE = float(_D) ** 0.5
_NC = 2
_NS = 16
_NW = _NC * _NS
_LANES = 16
_NB = 4


@functools.partial(jax.jit, static_argnums=(2, 3))
def _lookup(x_flat, table, per_w, chunk):
  n_chunks = per_w // chunk
  assert n_chunks % _NB == 0
  mesh = plsc.VectorSubcoreMesh(
      core_axis_name="c", subcore_axis_name="s", num_cores=_NC,
      num_subcores=_NS)

  @functools.partial(
      pl.kernel,
      mesh=mesh,
      out_type=jax.ShapeDtypeStruct((x_flat.shape[0], _D), jnp.float32),
      scratch_types=[
          pltpu.VMEM((per_w,), jnp.int32),
          [pltpu.VMEM((chunk, _D), jnp.float32) for _ in range(_NB)],
          [pltpu.SemaphoreType.DMA for _ in range(_NB)],
          [pltpu.SemaphoreType.DMA for _ in range(_NB)],
      ],
      compiler_params=pltpu.CompilerParams(use_tc_tiling_on_sc=False),
  )
  def body(x_hbm, table_hbm, out_hbm, idx_v, rows, sem_g, sem_s):
    wid = lax.axis_index("s") * _NC + lax.axis_index("c")
    base = wid * per_w
    pltpu.sync_copy(x_hbm.at[pl.ds(base, per_w)], idx_v)

    def gather(c, b):
      return pltpu.make_async_copy(
          table_hbm.at[idx_v.at[pl.ds(c * chunk, chunk)]], rows[b], sem_g[b])

    def scatter(c, b):
      return pltpu.make_async_copy(
          rows[b], out_hbm.at[pl.ds(base + c * chunk, chunk)], sem_s[b])

    for b in range(_NB - 1):
      gather(b, b).start()

    def group_body(go, carry):
      for b in range(_NB):
        c = go * _NB + b
        gather(c, b).wait()

        def scale_body(r, acc):
          for j in range(_D // _LANES):
            sl = pl.ds(j * _LANES, _LANES)
            rows[b][r, sl] = rows[b][r, sl] * _SCALE
          return acc

        lax.fori_loop(0, chunk, scale_body, 0, unroll=4)
        scatter(c, b).start()

        nb = (b + _NB - 1) % _NB
        nc = c + _NB - 1

        @pl.when(nc < n_chunks)
        def _():
          @pl.when(c >= 1)
          def _():
            scatter(c - 1, nb).wait()
          gather(nc, nb).start()

      return carry

    lax.fori_loop(0, n_chunks // _NB, group_body, 0)
    for b in range(_NB):
      scatter(n_chunks - _NB + b, b).wait()

  return body(x_flat, table)


def kernel(x, table):
  b, l = x.shape
  n = b * l
  per_w = n // _NW
  x_flat = jnp.reshape(x, (n,)).astype(jnp.int32)
  out = _lookup(x_flat, table, per_w, 256)
  return jnp.reshape(out, (b, l, _D))
